# baseline (speedup 1.0000x reference)
import jax
import jax.numpy as jnp
from jax import lax
from jax.experimental import pallas as pl
from jax.experimental.pallas import tpu as pltpu
from jax.experimental.pallas import tpu_sc as plsc

N_EDGES = 320000
N_SEG = 10000
N_SEG_PAD = 10112
D = 128
DH = D // 2

NC = 2
NS = 16

CHUNK = 256
ROWS_PER_CHUNK = CHUNK // 128
N_CHUNKS = N_EDGES // CHUNK
ITERS = (N_CHUNKS + NS - 1) // NS

SEG_PER_SUB = N_SEG_PAD // NS
CL = 8


def _sc_body(xh_hbm, meta_hbm, zs_hbm, zc_hbm, ones_hbm,
             psums_hbm, pcnts_hbm,
             xbufs, ridbufs, metabufs, obufs, onesbuf,
             acc_s, acc_c, gsems, ssems):
    c = lax.axis_index("c")
    s = lax.axis_index("s")

    iota16 = lax.iota(jnp.int32, 16)

    pltpu.sync_copy(ones_hbm, onesbuf)
    off = s * SEG_PER_SUB
    pltpu.sync_copy(zs_hbm, acc_s.at[pl.ds(off, SEG_PER_SUB)])
    pltpu.sync_copy(zc_hbm, acc_c.at[pl.ds(off, SEG_PER_SUB)])
    plsc.subcore_barrier()

    def chunk_t(i):
        return s + i * NS


    def issue_gathers(i, xb, ib):
        t = chunk_t(i)
        base = t * CHUNK
        rowb = t * 2 * ROWS_PER_CHUNK
        xbuf, ridbuf, metabuf = xbufs[xb], ridbufs[xb], metabufs[ib]

        def rid_body(g, gc):
            e0 = base + g * 16
            ridbuf[g // 8, pl.ds((g % 8) * 16, 16)] = 2 * (e0 + iota16) + c
            return gc

        lax.fori_loop(0, CHUNK // 16, rid_body, 0)

        pltpu.async_copy(meta_hbm.at[pl.ds(rowb, 2 * ROWS_PER_CHUNK)], metabuf,
                         gsems[xb])
        for j in range(ROWS_PER_CHUNK):
            pltpu.async_copy(xh_hbm.at[ridbuf.at[j]],
                             xbuf.at[pl.ds(j * 128, 128)], gsems[xb])

    def wait_gathers(i, xb, ib):
        t = chunk_t(i)
        rowb = t * 2 * ROWS_PER_CHUNK
        xbuf, ridbuf, metabuf = xbufs[xb], ridbufs[xb], metabufs[ib]
        pltpu.make_async_copy(meta_hbm.at[pl.ds(rowb, 2 * ROWS_PER_CHUNK)],
                              metabuf, gsems[xb]).wait()
        for j in range(ROWS_PER_CHUNK):
            pltpu.make_async_copy(xh_hbm.at[ridbuf.at[j]],
                                  xbuf.at[pl.ds(j * 128, 128)], gsems[xb]).wait()

    def process_chunk(xb, ib):
        xbuf, metabuf, obuf = xbufs[xb], metabufs[ib], obufs[xb]

        def group_body(g, gc):
            wvec = plsc.bitcast(
                metabuf[ROWS_PER_CHUNK + g // 8, pl.ds((g % 8) * 16, 16)],
                jnp.float32)
            for l in range(0, 16, 2):
                e0 = g * 16 + l
                e1 = e0 + 1
                w0 = wvec[l]
                w1 = wvec[l + 1]
                vals0 = [xbuf[e0, pl.ds(j * 16, 16)] for j in range(DH // 16)]
                vals1 = [xbuf[e1, pl.ds(j * 16, 16)] for j in range(DH // 16)]
                for j in range(DH // 16):
                    obuf[e0, pl.ds(j * 16, 16)] = vals0[j] * w0
                for j in range(DH // 16):
                    obuf[e1, pl.ds(j * 16, 16)] = vals1[j] * w1
            return gc

        lax.fori_loop(0, CHUNK // 16, group_body, 0)

        for j in range(ROWS_PER_CHUNK):
            pltpu.async_copy(obuf.at[pl.ds(j * 128, 128)],
                             acc_s.at[metabuf.at[j]], ssems[xb], add=True)

        @pl.when(c == xb)
        def _():
            for j in range(ROWS_PER_CHUNK):
                pltpu.async_copy(onesbuf, acc_c.at[metabuf.at[j]], ssems[xb],
                                 add=True)

    def drain_scatters(xb, ib):
        metabuf, obuf = metabufs[ib], obufs[xb]
        for j in range(ROWS_PER_CHUNK):
            pltpu.make_async_copy(obuf.at[pl.ds(j * 128, 128)],
                                  acc_s.at[metabuf.at[j]], ssems[xb]).wait()

        @pl.when(c == xb)
        def _():
            for j in range(ROWS_PER_CHUNK):
                pltpu.make_async_copy(onesbuf, acc_c.at[metabuf.at[j]],
                                      ssems[xb]).wait()


    def loop_body(jj, carry):
        for q in range(4):
            i = 4 * jj + q
            t = chunk_t(i)

            @pl.when((t >= s + 3 * NS) & (t - 3 * NS < N_CHUNKS))
            def _():
                drain_scatters((q + 1) % 2, (q + 1) % 4)

            @pl.when(t < N_CHUNKS)
            def _():
                issue_gathers(i, q % 2, q)

            @pl.when((i >= 1) & (t - NS < N_CHUNKS))
            def _():
                wait_gathers(i - 1, (q + 1) % 2, (q + 3) % 4)
                process_chunk((q + 1) % 2, (q + 3) % 4)

        return carry

    H4 = ITERS // 4 + 1
    lax.fori_loop(0, H4, loop_body, 0)

    for k in (4 * H4 - 3, 4 * H4 - 2):
        @pl.when(chunk_t(k) < N_CHUNKS)
        def _():
            drain_scatters(k % 2, k % 4)

    plsc.subcore_barrier()

    hoff = c * N_SEG_PAD + off
    pltpu.sync_copy(acc_s.at[pl.ds(off, SEG_PER_SUB)],
                    psums_hbm.at[pl.ds(hoff, SEG_PER_SUB)])

    pltpu.sync_copy(acc_c.at[pl.ds(off, SEG_PER_SUB)],
                    pcnts_hbm.at[pl.ds(hoff, SEG_PER_SUB)])


def _combine_body(p0, p1, c0, c1, o):
    cnt = jnp.sum(c0[...], axis=1) + jnp.sum(c1[...], axis=1)
    cnt = jnp.maximum(cnt, 1.0)
    o[...] = jnp.concatenate([p0[...], p1[...]], axis=1) / cnt[:, None]


BLK = 632


def kernel(x, index, weights):
    xh = x.reshape(2 * N_EDGES, DH)
    meta = jnp.concatenate(
        [index.reshape(N_CHUNKS, ROWS_PER_CHUNK, 128),
         lax.bitcast_convert_type(weights, jnp.int32).reshape(
             N_CHUNKS, ROWS_PER_CHUNK, 128)],
        axis=1).reshape(N_CHUNKS * 2 * ROWS_PER_CHUNK, 128)

    mesh = plsc.VectorSubcoreMesh(core_axis_name="c", subcore_axis_name="s")
    phase1 = pl.kernel(
        _sc_body,
        out_type=[
            jax.ShapeDtypeStruct((NC * N_SEG_PAD, DH), jnp.float32),
            jax.ShapeDtypeStruct((NC * N_SEG_PAD, CL), jnp.float32),
        ],
        mesh=mesh,
        compiler_params=pltpu.CompilerParams(use_tc_tiling_on_sc=False,
                                             needs_layout_passes=False),
        scratch_types=[
            [pltpu.VMEM((CHUNK, DH), jnp.float32) for _ in range(2)],
            [pltpu.VMEM((ROWS_PER_CHUNK, 128), jnp.int32) for _ in range(2)],
            [pltpu.VMEM((2 * ROWS_PER_CHUNK, 128), jnp.int32) for _ in range(4)],
            [pltpu.VMEM((CHUNK, DH), jnp.float32) for _ in range(2)],
            pltpu.VMEM((128, CL), jnp.float32),
            pltpu.VMEM_SHARED((N_SEG_PAD, DH), jnp.float32),
            pltpu.VMEM_SHARED((N_SEG_PAD, CL), jnp.float32),
            [pltpu.SemaphoreType.DMA for _ in range(2)],
            [pltpu.SemaphoreType.DMA for _ in range(2)],
        ],
    )
    zs = jnp.zeros((SEG_PER_SUB, DH), jnp.float32)
    zc = jnp.zeros((SEG_PER_SUB, CL), jnp.float32)
    ones = jnp.zeros((128, CL), jnp.float32).at[:, 0].set(1.0)
    psums, pcnts = phase1(xh, meta, zs, zc, ones)

    nblk = N_SEG_PAD // BLK
    out = pl.pallas_call(
        _combine_body,
        grid=(nblk,),
        in_specs=[
            pl.BlockSpec((BLK, DH), lambda i: (i, 0)),
            pl.BlockSpec((BLK, DH), lambda i: (i + nblk, 0)),
            pl.BlockSpec((BLK, CL), lambda i: (i, 0)),
            pl.BlockSpec((BLK, CL), lambda i: (i + nblk, 0)),
        ],
        out_specs=pl.BlockSpec((BLK, D), lambda i: (i, 0)),
        out_shape=jax.ShapeDtypeStruct((N_SEG_PAD, D), jnp.float32),
    )(psums, psums, pcnts, pcnts)
    return out[:N_SEG]


# speedup vs baseline: 1.0263x; 1.0263x over previous
import jax
import jax.numpy as jnp
from jax import lax
from jax.experimental import pallas as pl
from jax.experimental.pallas import tpu as pltpu
from jax.experimental.pallas import tpu_sc as plsc

N_EDGES = 320000
N_SEG = 10000
N_SEG_PAD = 10112
D = 128
DH = D // 2

NC = 2
NS = 16

CHUNK = 256
ROWS_PER_CHUNK = CHUNK // 128
N_CHUNKS = N_EDGES // CHUNK
ITERS = (N_CHUNKS + NS - 1) // NS

SEG_PER_SUB = N_SEG_PAD // NS
CL = 8


def _sc_body(xh_hbm, idx_hbm, w_hbm, zs_hbm, zc_hbm, ones_hbm,
             psums_hbm, pcnts_hbm,
             xbufs, ridbufs, idxbufs, wbufs, obufs, onesbuf,
             acc_s, acc_c, gsems, ssems):
    c = lax.axis_index("c")
    s = lax.axis_index("s")

    iota16 = lax.iota(jnp.int32, 16)

    pltpu.sync_copy(ones_hbm, onesbuf)
    off = s * SEG_PER_SUB
    pltpu.sync_copy(zs_hbm, acc_s.at[pl.ds(off, SEG_PER_SUB)])
    pltpu.sync_copy(zc_hbm, acc_c.at[pl.ds(off, SEG_PER_SUB)])
    plsc.subcore_barrier()

    def chunk_t(i):
        return s + i * NS


    def issue_gathers(i, xb, ib):
        t = chunk_t(i)
        base = t * CHUNK
        rowb = t * ROWS_PER_CHUNK
        xbuf, ridbuf, idxbuf, wbuf = xbufs[xb], ridbufs[xb], idxbufs[ib], wbufs[xb]

        def rid_body(g, gc):
            e0 = base + g * 16
            ridbuf[g // 8, pl.ds((g % 8) * 16, 16)] = 2 * (e0 + iota16) + c
            return gc

        lax.fori_loop(0, CHUNK // 16, rid_body, 0)

        pltpu.async_copy(idx_hbm.at[pl.ds(rowb, ROWS_PER_CHUNK)], idxbuf, gsems[xb])
        pltpu.async_copy(w_hbm.at[pl.ds(rowb, ROWS_PER_CHUNK)], wbuf, gsems[xb])
        for j in range(ROWS_PER_CHUNK):
            pltpu.async_copy(xh_hbm.at[ridbuf.at[j]],
                             xbuf.at[pl.ds(j * 128, 128)], gsems[xb])

    def wait_gathers(i, xb, ib):
        t = chunk_t(i)
        rowb = t * ROWS_PER_CHUNK
        xbuf, ridbuf, idxbuf, wbuf = xbufs[xb], ridbufs[xb], idxbufs[ib], wbufs[xb]
        pltpu.make_async_copy(idx_hbm.at[pl.ds(rowb, ROWS_PER_CHUNK)], idxbuf,
                              gsems[xb]).wait()
        pltpu.make_async_copy(w_hbm.at[pl.ds(rowb, ROWS_PER_CHUNK)], wbuf,
                              gsems[xb]).wait()
        for j in range(ROWS_PER_CHUNK):
            pltpu.make_async_copy(xh_hbm.at[ridbuf.at[j]],
                                  xbuf.at[pl.ds(j * 128, 128)], gsems[xb]).wait()

    def process_chunk(xb, ib):
        xbuf, idxbuf, wbuf, obuf = xbufs[xb], idxbufs[ib], wbufs[xb], obufs[xb]

        def group_body(g, gc):
            wvec = wbuf[g // 8, pl.ds((g % 8) * 16, 16)]
            for l in range(0, 16, 2):
                e0 = g * 16 + l
                e1 = e0 + 1
                w0 = wvec[l]
                w1 = wvec[l + 1]
                vals0 = [xbuf[e0, pl.ds(j * 16, 16)] for j in range(DH // 16)]
                vals1 = [xbuf[e1, pl.ds(j * 16, 16)] for j in range(DH // 16)]
                for j in range(DH // 16):
                    obuf[e0, pl.ds(j * 16, 16)] = vals0[j] * w0
                for j in range(DH // 16):
                    obuf[e1, pl.ds(j * 16, 16)] = vals1[j] * w1
            return gc

        lax.fori_loop(0, CHUNK // 16, group_body, 0)

        for j in range(ROWS_PER_CHUNK):
            pltpu.async_copy(obuf.at[pl.ds(j * 128, 128)],
                             acc_s.at[idxbuf.at[j]], ssems[xb], add=True)

        @pl.when(c == xb)
        def _():
            for j in range(ROWS_PER_CHUNK):
                pltpu.async_copy(onesbuf, acc_c.at[idxbuf.at[j]], ssems[xb],
                                 add=True)

    def drain_scatters(xb, ib):
        idxbuf, obuf = idxbufs[ib], obufs[xb]
        for j in range(ROWS_PER_CHUNK):
            pltpu.make_async_copy(obuf.at[pl.ds(j * 128, 128)],
                                  acc_s.at[idxbuf.at[j]], ssems[xb]).wait()

        @pl.when(c == xb)
        def _():
            for j in range(ROWS_PER_CHUNK):
                pltpu.make_async_copy(onesbuf, acc_c.at[idxbuf.at[j]],
                                      ssems[xb]).wait()


    def loop_body(jj, carry):
        for q in range(4):
            i = 4 * jj + q
            t = chunk_t(i)

            @pl.when((t >= s + 3 * NS) & (t - 3 * NS < N_CHUNKS))
            def _():
                drain_scatters((q + 1) % 2, (q + 1) % 4)

            @pl.when(t < N_CHUNKS)
            def _():
                issue_gathers(i, q % 2, q)

            @pl.when((i >= 1) & (t - NS < N_CHUNKS))
            def _():
                wait_gathers(i - 1, (q + 1) % 2, (q + 3) % 4)
                process_chunk((q + 1) % 2, (q + 3) % 4)

        return carry

    H4 = ITERS // 4 + 1
    lax.fori_loop(0, H4, loop_body, 0)

    for k in (4 * H4 - 3, 4 * H4 - 2):
        @pl.when(chunk_t(k) < N_CHUNKS)
        def _():
            drain_scatters(k % 2, k % 4)

    plsc.subcore_barrier()

    hoff = c * N_SEG_PAD + off
    pltpu.sync_copy(acc_s.at[pl.ds(off, SEG_PER_SUB)],
                    psums_hbm.at[pl.ds(hoff, SEG_PER_SUB)])

    pltpu.sync_copy(acc_c.at[pl.ds(off, SEG_PER_SUB)],
                    pcnts_hbm.at[pl.ds(hoff, SEG_PER_SUB)])


def _combine_body(p0, p1, c0, c1, o):
    cnt = jnp.sum(c0[...], axis=1) + jnp.sum(c1[...], axis=1)
    cnt = jnp.maximum(cnt, 1.0)
    o[...] = jnp.concatenate([p0[...], p1[...]], axis=1) / cnt[:, None]


BLK = 632


def kernel(x, index, weights):
    xh = x.reshape(2 * N_EDGES, DH)
    idx2d = index.reshape(N_EDGES // 128, 128)
    w2d = weights.reshape(N_EDGES // 128, 128)

    mesh = plsc.VectorSubcoreMesh(core_axis_name="c", subcore_axis_name="s")
    phase1 = pl.kernel(
        _sc_body,
        out_type=[
            jax.ShapeDtypeStruct((NC * N_SEG_PAD, DH), jnp.float32),
            jax.ShapeDtypeStruct((NC * N_SEG_PAD, CL), jnp.float32),
        ],
        mesh=mesh,
        compiler_params=pltpu.CompilerParams(use_tc_tiling_on_sc=False),
        scratch_types=[
            [pltpu.VMEM((CHUNK, DH), jnp.float32) for _ in range(2)],
            [pltpu.VMEM((ROWS_PER_CHUNK, 128), jnp.int32) for _ in range(2)],
            [pltpu.VMEM((ROWS_PER_CHUNK, 128), jnp.int32) for _ in range(4)],
            [pltpu.VMEM((ROWS_PER_CHUNK, 128), jnp.float32) for _ in range(2)],
            [pltpu.VMEM((CHUNK, DH), jnp.float32) for _ in range(2)],
            pltpu.VMEM((128, CL), jnp.float32),
            pltpu.VMEM_SHARED((N_SEG_PAD, DH), jnp.float32),
            pltpu.VMEM_SHARED((N_SEG_PAD, CL), jnp.float32),
            [pltpu.SemaphoreType.DMA for _ in range(2)],
            [pltpu.SemaphoreType.DMA for _ in range(2)],
        ],
    )
    zs = jnp.zeros((SEG_PER_SUB, DH), jnp.float32)
    zc = jnp.zeros((SEG_PER_SUB, CL), jnp.float32)
    ones = jnp.zeros((128, CL), jnp.float32).at[:, 0].set(1.0)
    psums, pcnts = phase1(xh, idx2d, w2d, zs, zc, ones)

    nblk = N_SEG_PAD // BLK
    out = pl.pallas_call(
        _combine_body,
        grid=(nblk,),
        in_specs=[
            pl.BlockSpec((BLK, DH), lambda i: (i, 0)),
            pl.BlockSpec((BLK, DH), lambda i: (i + nblk, 0)),
            pl.BlockSpec((BLK, CL), lambda i: (i, 0)),
            pl.BlockSpec((BLK, CL), lambda i: (i + nblk, 0)),
        ],
        out_specs=pl.BlockSpec((BLK, D), lambda i: (i, 0)),
        out_shape=jax.ShapeDtypeStruct((N_SEG_PAD, D), jnp.float32),
    )(psums, psums, pcnts, pcnts)
    return out[:N_SEG]


# speedup vs baseline: 1.0463x; 1.0195x over previous
import jax
import jax.numpy as jnp
from jax import lax
from jax.experimental import pallas as pl
from jax.experimental.pallas import tpu as pltpu
from jax.experimental.pallas import tpu_sc as plsc

N_EDGES = 320000
N_SEG = 10000
N_SEG_PAD = 10112
D = 128
DH = D // 2

NC = 2
NS = 16

CHUNK = 256
ROWS_PER_CHUNK = CHUNK // 128
N_CHUNKS = N_EDGES // CHUNK
ITERS = (N_CHUNKS + NS - 1) // NS

SEG_PER_SUB = N_SEG_PAD // NS
CL = 8


def _sc_body(xh_hbm, idx_hbm, w_hbm, zs_hbm, zc_hbm, ones_hbm,
             psums_hbm, pcnts_hbm,
             xbufs, ridbufs, idxbufs, wbufs, obufs, onesbuf,
             acc_s, acc_c, gsems, ssems):
    c = lax.axis_index("c")
    s = lax.axis_index("s")

    iota16 = lax.iota(jnp.int32, 16)

    pltpu.sync_copy(ones_hbm, onesbuf)
    off = s * SEG_PER_SUB
    pltpu.sync_copy(zs_hbm, acc_s.at[pl.ds(off, SEG_PER_SUB)])
    pltpu.sync_copy(zc_hbm, acc_c.at[pl.ds(off, SEG_PER_SUB)])
    plsc.subcore_barrier()

    def chunk_t(i):
        return s + i * NS


    def issue_gathers(i, xb, ib):
        t = chunk_t(i)
        base = t * CHUNK
        rowb = t * ROWS_PER_CHUNK
        xbuf, ridbuf, idxbuf, wbuf = xbufs[xb], ridbufs[xb], idxbufs[ib], wbufs[xb]

        def rid_body(g, gc):
            e0 = base + g * 16
            ridbuf[g // 8, pl.ds((g % 8) * 16, 16)] = 2 * (e0 + iota16) + c
            return gc

        lax.fori_loop(0, CHUNK // 16, rid_body, 0)

        pltpu.async_copy(idx_hbm.at[pl.ds(rowb, ROWS_PER_CHUNK)], idxbuf, gsems[xb])
        pltpu.async_copy(w_hbm.at[pl.ds(rowb, ROWS_PER_CHUNK)], wbuf, gsems[xb])
        for j in range(ROWS_PER_CHUNK):
            pltpu.async_copy(xh_hbm.at[ridbuf.at[j]],
                             xbuf.at[pl.ds(j * 128, 128)], gsems[xb])

    def wait_gathers(i, xb, ib):
        t = chunk_t(i)
        rowb = t * ROWS_PER_CHUNK
        xbuf, ridbuf, idxbuf, wbuf = xbufs[xb], ridbufs[xb], idxbufs[ib], wbufs[xb]
        pltpu.make_async_copy(idx_hbm.at[pl.ds(rowb, ROWS_PER_CHUNK)], idxbuf,
                              gsems[xb]).wait()
        pltpu.make_async_copy(w_hbm.at[pl.ds(rowb, ROWS_PER_CHUNK)], wbuf,
                              gsems[xb]).wait()
        for j in range(ROWS_PER_CHUNK):
            pltpu.make_async_copy(xh_hbm.at[ridbuf.at[j]],
                                  xbuf.at[pl.ds(j * 128, 128)], gsems[xb]).wait()

    def process_chunk(xb, ib):
        xbuf, idxbuf, wbuf, obuf = xbufs[xb], idxbufs[ib], wbufs[xb], obufs[xb]

        def group_body(g, gc):
            wvec = wbuf[g // 8, pl.ds((g % 8) * 16, 16)]
            for l in range(0, 16, 2):
                e0 = g * 16 + l
                e1 = e0 + 1
                w0 = wvec[l]
                w1 = wvec[l + 1]
                vals0 = [xbuf[e0, pl.ds(j * 16, 16)] for j in range(DH // 16)]
                vals1 = [xbuf[e1, pl.ds(j * 16, 16)] for j in range(DH // 16)]
                for j in range(DH // 16):
                    obuf[e0, pl.ds(j * 16, 16)] = vals0[j] * w0
                for j in range(DH // 16):
                    obuf[e1, pl.ds(j * 16, 16)] = vals1[j] * w1
            return gc

        lax.fori_loop(0, CHUNK // 16, group_body, 0)

        for j in range(ROWS_PER_CHUNK):
            pltpu.async_copy(obuf.at[pl.ds(j * 128, 128)],
                             acc_s.at[idxbuf.at[j]], ssems[xb], add=True)

        @pl.when(c == 0)
        def _():
            for j in range(ROWS_PER_CHUNK):
                pltpu.async_copy(onesbuf, acc_c.at[idxbuf.at[j]], ssems[xb],
                                 add=True)

    def drain_scatters(xb, ib):
        idxbuf, obuf = idxbufs[ib], obufs[xb]
        for j in range(ROWS_PER_CHUNK):
            pltpu.make_async_copy(obuf.at[pl.ds(j * 128, 128)],
                                  acc_s.at[idxbuf.at[j]], ssems[xb]).wait()

        @pl.when(c == 0)
        def _():
            for j in range(ROWS_PER_CHUNK):
                pltpu.make_async_copy(onesbuf, acc_c.at[idxbuf.at[j]],
                                      ssems[xb]).wait()


    def loop_body(jj, carry):
        for q in range(4):
            i = 4 * jj + q
            t = chunk_t(i)

            @pl.when((t >= s + 3 * NS) & (t - 3 * NS < N_CHUNKS))
            def _():
                drain_scatters((q + 1) % 2, (q + 1) % 4)

            @pl.when(t < N_CHUNKS)
            def _():
                issue_gathers(i, q % 2, q)

            @pl.when((i >= 1) & (t - NS < N_CHUNKS))
            def _():
                wait_gathers(i - 1, (q + 1) % 2, (q + 3) % 4)
                process_chunk((q + 1) % 2, (q + 3) % 4)

        return carry

    H4 = ITERS // 4 + 1
    lax.fori_loop(0, H4, loop_body, 0)

    for k in (4 * H4 - 3, 4 * H4 - 2):
        @pl.when(chunk_t(k) < N_CHUNKS)
        def _():
            drain_scatters(k % 2, k % 4)

    plsc.subcore_barrier()

    hoff = c * N_SEG_PAD + off
    pltpu.sync_copy(acc_s.at[pl.ds(off, SEG_PER_SUB)],
                    psums_hbm.at[pl.ds(hoff, SEG_PER_SUB)])

    @pl.when(c == 0)
    def _():
        pltpu.sync_copy(acc_c.at[pl.ds(off, SEG_PER_SUB)],
                        pcnts_hbm.at[pl.ds(off, SEG_PER_SUB)])


def _combine_body(p0, p1, cn, o):
    cnt = jnp.sum(cn[...], axis=1)
    cnt = jnp.maximum(cnt, 1.0)
    o[...] = jnp.concatenate([p0[...], p1[...]], axis=1) / cnt[:, None]


BLK = 632


def kernel(x, index, weights):
    xh = x.reshape(2 * N_EDGES, DH)
    idx2d = index.reshape(N_EDGES // 128, 128)
    w2d = weights.reshape(N_EDGES // 128, 128)

    mesh = plsc.VectorSubcoreMesh(core_axis_name="c", subcore_axis_name="s")
    phase1 = pl.kernel(
        _sc_body,
        out_type=[
            jax.ShapeDtypeStruct((NC * N_SEG_PAD, DH), jnp.float32),
            jax.ShapeDtypeStruct((N_SEG_PAD, CL), jnp.float32),
        ],
        mesh=mesh,
        compiler_params=pltpu.CompilerParams(use_tc_tiling_on_sc=False),
        scratch_types=[
            [pltpu.VMEM((CHUNK, DH), jnp.float32) for _ in range(2)],
            [pltpu.VMEM((ROWS_PER_CHUNK, 128), jnp.int32) for _ in range(2)],
            [pltpu.VMEM((ROWS_PER_CHUNK, 128), jnp.int32) for _ in range(4)],
            [pltpu.VMEM((ROWS_PER_CHUNK, 128), jnp.float32) for _ in range(2)],
            [pltpu.VMEM((CHUNK, DH), jnp.float32) for _ in range(2)],
            pltpu.VMEM((128, CL), jnp.float32),
            pltpu.VMEM_SHARED((N_SEG_PAD, DH), jnp.float32),
            pltpu.VMEM_SHARED((N_SEG_PAD, CL), jnp.float32),
            [pltpu.SemaphoreType.DMA for _ in range(2)],
            [pltpu.SemaphoreType.DMA for _ in range(2)],
        ],
    )
    zs = jnp.zeros((SEG_PER_SUB, DH), jnp.float32)
    zc = jnp.zeros((SEG_PER_SUB, CL), jnp.float32)
    ones = jnp.zeros((128, CL), jnp.float32).at[:, 0].set(1.0)
    psums, pcnts = phase1(xh, idx2d, w2d, zs, zc, ones)

    nblk = N_SEG_PAD // BLK
    out = pl.pallas_call(
        _combine_body,
        grid=(nblk,),
        in_specs=[
            pl.BlockSpec((BLK, DH), lambda i: (i, 0)),
            pl.BlockSpec((BLK, DH), lambda i: (i + nblk, 0)),
            pl.BlockSpec((BLK, CL), lambda i: (i, 0)),
        ],
        out_specs=pl.BlockSpec((BLK, D), lambda i: (i, 0)),
        out_shape=jax.ShapeDtypeStruct((N_SEG_PAD, D), jnp.float32),
    )(psums, psums, pcnts)
    return out[:N_SEG]
